# Initial kernel scaffold; baseline (speedup 1.0000x reference)
import jax
import jax.numpy as jnp
from jax.experimental import pallas as pl


def kernel(x, edge_index, W_gcn, b_gcn, bn1_gamma, bn1_beta, W_ff1, b_ff1, W_ff2, b_ff2, bn2_gamma, bn2_beta):
    raise NotImplementedError("write your pallas kernel here")


# speedup vs baseline: 13.2381x; 13.2381x over previous
import functools

import jax
import jax.numpy as jnp
from jax import lax
from jax.experimental import pallas as pl
from jax.experimental.pallas import tpu as pltpu
from jax.experimental.pallas import tpu_sc as plsc

_N = 10000
_D = 128
_E = 320000
_EPS = 1e-5

_NC = 2
_NS = 16
_NW = _NC * _NS
_K = 128
_CPW = 79
_TOTCH = _NW * _CPW
_EPAD = _TOTCH * _K
_NPAD = 10240
_RPS = _NPAD // _NS
_DEGW = 128

_sc_mesh = plsc.VectorSubcoreMesh(core_axis_name="c", subcore_axis_name="s")


def _deg_body(col_hbm, zeros_hbm, out_hbm, idx_v, ones_v, acc):
    c = lax.axis_index("c")
    s = lax.axis_index("s")
    wid = c * _NS + s
    pltpu.sync_copy(zeros_hbm.at[pl.ds(s * _RPS, _RPS)],
                    acc.at[pl.ds(s * _RPS, _RPS)])
    lane = lax.broadcasted_iota(jnp.int32, (16,), 0)
    pat = jnp.where(lane == 0, 1.0, 0.0).astype(jnp.float32)
    zv = jnp.zeros((16,), jnp.float32)

    def fill(j, carry):
        ones_v[j, pl.ds(0, 16)] = pat
        for l in range(1, _DEGW // 16):
            ones_v[j, pl.ds(l * 16, 16)] = zv
        return carry

    lax.fori_loop(0, _K, fill, 0)
    plsc.subcore_barrier()

    def body(j, carry):
        pltpu.sync_copy(col_hbm.at[wid * _CPW + j], idx_v)
        pltpu.sync_copy(ones_v, acc.at[idx_v], add=True)
        return carry

    lax.fori_loop(0, _CPW, body, 0)
    plsc.subcore_barrier()
    pltpu.sync_copy(acc.at[pl.ds(s * _RPS, _RPS)],
                    out_hbm.at[c, pl.ds(s * _RPS, _RPS)])


_deg_call = pl.kernel(
    _deg_body,
    out_type=jax.ShapeDtypeStruct((_NC, _NPAD, _D), jnp.float32),
    mesh=_sc_mesh,
    scratch_types=[
        pltpu.VMEM((_K,), jnp.int32),
        pltpu.VMEM((_K, _DEGW), jnp.float32),
        pltpu.VMEM_SHARED((_NPAD, _DEGW), jnp.float32),
    ],
)


def _agg_body(row_hbm, col_hbm, y_hbm, zeros_hbm, out_hbm,
              idx_r, idx_c, rows_v, acc, sem):
    c = lax.axis_index("c")
    s = lax.axis_index("s")
    wid = c * _NS + s
    pltpu.sync_copy(zeros_hbm.at[pl.ds(s * _RPS, _RPS)],
                    acc.at[pl.ds(s * _RPS, _RPS)])
    plsc.subcore_barrier()

    def body(j, carry):
        pltpu.sync_copy(row_hbm.at[wid * _CPW + j], idx_r)
        pltpu.sync_copy(col_hbm.at[wid * _CPW + j], idx_c)
        pltpu.async_copy(y_hbm.at[idx_r], rows_v, sem).wait()
        pltpu.sync_copy(rows_v, acc.at[idx_c], add=True)
        return carry

    lax.fori_loop(0, _CPW, body, 0)
    plsc.subcore_barrier()
    pltpu.sync_copy(acc.at[pl.ds(s * _RPS, _RPS)],
                    out_hbm.at[c, pl.ds(s * _RPS, _RPS)])


_agg_call = pl.kernel(
    _agg_body,
    out_type=jax.ShapeDtypeStruct((_NC, _NPAD, _D), jnp.float32),
    mesh=_sc_mesh,
    scratch_types=[
        pltpu.VMEM((_K,), jnp.int32),
        pltpu.VMEM((_K,), jnp.int32),
        pltpu.VMEM((_K, _D), jnp.float32),
        pltpu.VMEM_SHARED((_NPAD, _D), jnp.float32),
        pltpu.SemaphoreType.DMA,
    ],
)


def _prep_body(x_ref, w_ref, d0_ref, d1_ref, y_ref):
    deg = (jnp.sum(d0_ref[...], axis=1, keepdims=True)
           + jnp.sum(d1_ref[...], axis=1, keepdims=True) + 1.0)
    dis = lax.rsqrt(deg)
    xw = jnp.dot(x_ref[...], w_ref[...], preferred_element_type=jnp.float32)
    y_ref[...] = xw * dis


_prep_call = pl.pallas_call(
    _prep_body,
    out_shape=jax.ShapeDtypeStruct((_N, _D), jnp.float32),
)


def _post_body(x_ref, y_ref, a0_ref, a1_ref, d0_ref, d1_ref, bgcn_ref,
               g1_ref, b1_ref, wf1_ref, bf1_ref, wf2_ref, bf2_ref,
               g2_ref, b2_ref, out_ref):
    deg = (jnp.sum(d0_ref[...], axis=1, keepdims=True)
           + jnp.sum(d1_ref[...], axis=1, keepdims=True) + 1.0)
    dis = lax.rsqrt(deg)
    agg = a0_ref[...] + a1_ref[...] + y_ref[...]
    t = x_ref[...] + bgcn_ref[...] + dis * agg
    mean = jnp.mean(t, axis=0, keepdims=True)
    var = jnp.mean((t - mean) * (t - mean), axis=0, keepdims=True)
    h1 = g1_ref[...] * (t - mean) * lax.rsqrt(var + _EPS) + b1_ref[...]
    ff = jnp.maximum(
        jnp.dot(h1, wf1_ref[...], preferred_element_type=jnp.float32)
        + bf1_ref[...], 0.0)
    u = h1 + jnp.dot(ff, wf2_ref[...],
                     preferred_element_type=jnp.float32) + bf2_ref[...]
    mean2 = jnp.mean(u, axis=0, keepdims=True)
    var2 = jnp.mean((u - mean2) * (u - mean2), axis=0, keepdims=True)
    out_ref[...] = (g2_ref[...] * (u - mean2) * lax.rsqrt(var2 + _EPS)
                    + b2_ref[...])


_post_call = pl.pallas_call(
    _post_body,
    out_shape=jax.ShapeDtypeStruct((_N, _D), jnp.float32),
)


def kernel(x, edge_index, W_gcn, b_gcn, bn1_gamma, bn1_beta,
           W_ff1, b_ff1, W_ff2, b_ff2, bn2_gamma, bn2_beta):
    row = edge_index[0]
    col = edge_index[1]
    pad = _EPAD - _E
    sink = _N + (jnp.arange(pad, dtype=jnp.int32) % (_NPAD - _N))
    rowp = jnp.concatenate([row, jnp.zeros((pad,), jnp.int32)]).reshape(
        _TOTCH, _K)
    colp = jnp.concatenate([col, sink]).reshape(_TOTCH, _K)
    zeros_acc = jnp.zeros((_NPAD, _D), jnp.float32)

    degp = _deg_call(colp, zeros_acc)
    d0 = degp[0, :_N, :16]
    d1 = degp[1, :_N, :16]

    y = _prep_call(x, W_gcn, d0, d1)

    aggp = _agg_call(rowp, colp, y, zeros_acc)
    a0 = aggp[0, :_N]
    a1 = aggp[1, :_N]

    return _post_call(
        x, y, a0, a1, d0, d1,
        b_gcn.reshape(1, _D),
        bn1_gamma.reshape(1, _D), bn1_beta.reshape(1, _D),
        W_ff1, b_ff1.reshape(1, 2 * _D),
        W_ff2, b_ff2.reshape(1, _D),
        bn2_gamma.reshape(1, _D), bn2_beta.reshape(1, _D),
    )
